# baseline (device time: 544338 ns/iter reference)
import jax
import jax.numpy as jnp
import numpy as np
from jax import lax
from jax.experimental import pallas as pl
from jax.experimental.pallas import tpu as pltpu

N_DEV = 16
SQ = 2048
SKV = 2048
D_MODEL = 1024
H_LOC = 8
DH = 128
MB = 64
N_MB = SQ // MB
CHUNK = SQ // N_DEV
SCALE = 0.08838834764831843

_QB_GROUPS = [[b for b in range(N_MB) if b % 3 == r] for r in range(3)]
_ROW_PERM = np.concatenate(
    [np.arange(b * MB, (b + 1) * MB) for g in _QB_GROUPS for b in g]
)
_KCOL_PERM = _ROW_PERM
_G_OFF = [0, 704, 1408]
_G_LEN = [704, 704, 640]
_GROUPS = [
    (0, 11, _G_OFF[0], _G_LEN[0], None),
    (704, 11, _G_OFF[2], _G_LEN[2], _G_OFF[1]),
    (1408, 10, _G_OFF[1], _G_LEN[1], _G_OFF[2]),
]


def _attn_body(x_ref, wq_ref, k_ref, v_ref, wo_ref, out_ref):
    q = jnp.dot(x_ref[...], wq_ref[...], preferred_element_type=jnp.float32)

    for r, (row_off, nb, m_off, m_len, d_off) in enumerate(_GROUPS):
        nr = nb * MB
        acc = jnp.zeros((nr, D_MODEL), jnp.float32)
        for h in range(H_LOC):
            qh = q[row_off:row_off + nr, h * DH:(h + 1) * DH]
            s_main = lax.dot_general(
                qh, k_ref[h, m_off:m_off + m_len],
                (((1,), (1,)), ((), ())),
                preferred_element_type=jnp.float32,
            ) * SCALE
            if d_off is None:
                s = s_main
            else:
                s0 = lax.dot_general(
                    qh, k_ref[h, 0:MB], (((1,), (1,)), ((), ())),
                    preferred_element_type=jnp.float32,
                ) * SCALE
                qd = qh.reshape(nb, MB, DH)
                kd = k_ref[h, d_off:d_off + nr].reshape(nb, MB, DH)
                sd = lax.dot_general(
                    qd, kd, (((2,), (2,)), ((0,), (0,))),
                    preferred_element_type=jnp.float32,
                ) * SCALE
                s = jnp.concatenate(
                    [s_main, s0, sd.reshape(nr, MB)], axis=1
                )
            m = jnp.max(s, axis=1, keepdims=True)
            w = jnp.exp(s - m)
            w = w / jnp.sum(w, axis=1, keepdims=True)
            ctx = jnp.dot(
                w[:, :m_len], v_ref[h, m_off:m_off + m_len],
                preferred_element_type=jnp.float32,
            )
            if d_off is not None:
                ctx = ctx + jnp.dot(
                    w[:, m_len:m_len + MB], v_ref[h, 0:MB],
                    preferred_element_type=jnp.float32,
                )
                wd = w[:, m_len + MB:].reshape(nb, MB, MB)
                vd = v_ref[h, d_off:d_off + nr].reshape(nb, MB, DH)
                ctx = ctx + lax.dot_general(
                    wd, vd, (((2,), (1,)), ((0,), (0,))),
                    preferred_element_type=jnp.float32,
                ).reshape(nr, DH)
            acc = acc + jnp.dot(
                ctx, wo_ref[h], preferred_element_type=jnp.float32
            )
        for j, qb in enumerate(_QB_GROUPS[r]):
            out_ref[qb * MB:(qb + 1) * MB, :] = acc[j * MB:(j + 1) * MB, :]


def _allreduce_body(p_ref, out_ref, recv_ref, rs_sems, ag_sems, send_sem):
    d = lax.axis_index("i")
    left = (d - 1) % N_DEV
    right = (d + 1) % N_DEV

    barrier_sem = pltpu.get_barrier_semaphore()
    for nbr in [left, right]:
        pl.semaphore_signal(
            barrier_sem, inc=1,
            device_id=(nbr,), device_id_type=pl.DeviceIdType.MESH,
        )
    pl.semaphore_wait(barrier_sem, 2)

    out_ref[...] = p_ref[...]

    for h in range(N_DEV - 1):
        s = (d - h) % N_DEV
        rdma = pltpu.make_async_remote_copy(
            src_ref=out_ref.at[s],
            dst_ref=recv_ref.at[h],
            send_sem=send_sem,
            recv_sem=rs_sems.at[h],
            device_id=(right,),
            device_id_type=pl.DeviceIdType.MESH,
        )
        rdma.start()
        rdma.wait()
        sl = (d - 1 - h) % N_DEV
        out_ref[sl] = out_ref[sl] + recv_ref[h]

    for h in range(N_DEV - 1):
        s = (d + 1 - h) % N_DEV
        rdma = pltpu.make_async_remote_copy(
            src_ref=out_ref.at[s],
            dst_ref=out_ref.at[s],
            send_sem=send_sem,
            recv_sem=ag_sems.at[h],
            device_id=(right,),
            device_id_type=pl.DeviceIdType.MESH,
        )
        rdma.start()
        rdma.wait()


def kernel(x, Wq, K_ext, V_ext, Wo):
    d = lax.axis_index("i")

    x2 = x.reshape(SQ, D_MODEL)[_ROW_PERM]
    wq_loc = lax.dynamic_slice(Wq, (0, d * H_LOC * DH), (D_MODEL, H_LOC * DH))
    wo_loc = lax.dynamic_slice(
        Wo, (d * H_LOC * DH, 0), (H_LOC * DH, D_MODEL)
    ).reshape(H_LOC, DH, D_MODEL)
    k_loc = K_ext.reshape(SKV, H_LOC, DH).transpose(1, 0, 2)[:, _KCOL_PERM]
    v_loc = V_ext.reshape(SKV, H_LOC, DH).transpose(1, 0, 2)[:, _KCOL_PERM]

    partial = pl.pallas_call(
        _attn_body,
        in_specs=[pl.BlockSpec(memory_space=pltpu.VMEM)] * 5,
        out_specs=pl.BlockSpec(memory_space=pltpu.VMEM),
        out_shape=jax.ShapeDtypeStruct((SQ, D_MODEL), jnp.float32),
        compiler_params=pltpu.CompilerParams(
            vmem_limit_bytes=100 * 1024 * 1024
        ),
    )(x2, wq_loc, k_loc, v_loc, wo_loc)

    reduced = pl.pallas_call(
        _allreduce_body,
        in_specs=[pl.BlockSpec(memory_space=pltpu.VMEM)],
        out_specs=pl.BlockSpec(memory_space=pltpu.VMEM),
        out_shape=jax.ShapeDtypeStruct((N_DEV, CHUNK, D_MODEL), jnp.float32),
        scratch_shapes=[
            pltpu.VMEM((N_DEV - 1, CHUNK, D_MODEL), jnp.float32),
            pltpu.SemaphoreType.DMA((N_DEV - 1,)),
            pltpu.SemaphoreType.DMA((N_DEV - 1,)),
            pltpu.SemaphoreType.DMA,
        ],
        compiler_params=pltpu.CompilerParams(collective_id=0),
    )(partial.reshape(N_DEV, CHUNK, D_MODEL))

    return reduced.reshape(1, SQ, D_MODEL)


# device time: 333037 ns/iter; 1.6345x vs baseline; 1.6345x over previous
import jax
import jax.numpy as jnp
import numpy as np
from jax import lax
from jax.experimental import pallas as pl
from jax.experimental.pallas import tpu as pltpu

N_DEV = 16
SQ = 2048
SKV = 2048
D_MODEL = 1024
H_LOC = 8
DH = 128
MB = 64
N_MB = SQ // MB
CHUNK = SQ // N_DEV
SCALE = 0.08838834764831843

_QB_GROUPS = [[b for b in range(N_MB) if b % 3 == r] for r in range(3)]
_ROW_PERM = np.concatenate(
    [np.arange(b * MB, (b + 1) * MB) for g in _QB_GROUPS for b in g]
)
_KCOL_PERM = _ROW_PERM
_G_OFF = [0, 704, 1408]
_G_LEN = [704, 704, 640]
_GROUPS = [
    (0, 11, _G_OFF[0], _G_LEN[0], None),
    (704, 11, _G_OFF[2], _G_LEN[2], _G_OFF[1]),
    (1408, 10, _G_OFF[1], _G_LEN[1], _G_OFF[2]),
]


def _attn_body(x_ref, wq_ref, k_ref, v_ref, wo_ref, out_ref):
    q = jnp.dot(x_ref[...], wq_ref[...], preferred_element_type=jnp.float32)

    for r, (row_off, nb, m_off, m_len, d_off) in enumerate(_GROUPS):
        nr = nb * MB
        acc = jnp.zeros((nr, D_MODEL), jnp.float32)
        for h in range(H_LOC):
            qh = q[row_off:row_off + nr, h * DH:(h + 1) * DH]
            s_main = lax.dot_general(
                qh, k_ref[h, m_off:m_off + m_len],
                (((1,), (1,)), ((), ())),
                preferred_element_type=jnp.float32,
            ) * SCALE
            if d_off is None:
                s = s_main
            else:
                s0 = lax.dot_general(
                    qh, k_ref[h, 0:MB], (((1,), (1,)), ((), ())),
                    preferred_element_type=jnp.float32,
                ) * SCALE
                qd = qh.reshape(nb, MB, DH)
                kd = k_ref[h, d_off:d_off + nr].reshape(nb, MB, DH)
                sd = lax.dot_general(
                    qd, kd, (((2,), (2,)), ((0,), (0,))),
                    preferred_element_type=jnp.float32,
                ) * SCALE
                s = jnp.concatenate(
                    [s_main, s0, sd.reshape(nr, MB)], axis=1
                )
            m = jnp.max(s, axis=1, keepdims=True)
            w = jnp.exp(s - m)
            w = w / jnp.sum(w, axis=1, keepdims=True)
            ctx = jnp.dot(
                w[:, :m_len], v_ref[h, m_off:m_off + m_len],
                preferred_element_type=jnp.float32,
            )
            if d_off is not None:
                ctx = ctx + jnp.dot(
                    w[:, m_len:m_len + MB], v_ref[h, 0:MB],
                    preferred_element_type=jnp.float32,
                )
                wd = w[:, m_len + MB:].reshape(nb, MB, MB)
                vd = v_ref[h, d_off:d_off + nr].reshape(nb, MB, DH)
                ctx = ctx + lax.dot_general(
                    wd, vd, (((2,), (1,)), ((0,), (0,))),
                    preferred_element_type=jnp.float32,
                ).reshape(nr, DH)
            acc = acc + jnp.dot(
                ctx, wo_ref[h], preferred_element_type=jnp.float32
            )
        for j, qb in enumerate(_QB_GROUPS[r]):
            out_ref[qb * MB:(qb + 1) * MB, :] = acc[j * MB:(j + 1) * MB, :]


def _allreduce_body(p_ref, out_ref, recv_ref, rs_sems, ag_sems, send_sem):
    d = lax.axis_index("i")
    left = (d - 1) % N_DEV
    right = (d + 1) % N_DEV

    barrier_sem = pltpu.get_barrier_semaphore()
    for nbr in [left, right]:
        pl.semaphore_signal(
            barrier_sem, inc=1,
            device_id=(nbr,), device_id_type=pl.DeviceIdType.MESH,
        )
    pl.semaphore_wait(barrier_sem, 2)

    out_ref[...] = p_ref[...]

    for h in range(N_DEV - 1):
        s = (d - h) % N_DEV
        rdma = pltpu.make_async_remote_copy(
            src_ref=out_ref.at[s],
            dst_ref=recv_ref.at[h],
            send_sem=send_sem,
            recv_sem=rs_sems.at[h],
            device_id=(right,),
            device_id_type=pl.DeviceIdType.MESH,
        )
        rdma.start()
        rdma.wait()
        sl = (d - 1 - h) % N_DEV
        out_ref[sl] = out_ref[sl] + recv_ref[h]

    for h in range(N_DEV - 1):
        s = (d + 1 - h) % N_DEV
        rdma = pltpu.make_async_remote_copy(
            src_ref=out_ref.at[s],
            dst_ref=out_ref.at[s],
            send_sem=send_sem,
            recv_sem=ag_sems.at[h],
            device_id=(right,),
            device_id_type=pl.DeviceIdType.MESH,
        )
        rdma.start()
        rdma.wait()


def kernel(x, Wq, K_ext, V_ext, Wo):
    d = lax.axis_index("i")

    xb = x.reshape(N_MB, MB, D_MODEL)
    x2 = jnp.concatenate([xb[0::3], xb[1::3], xb[2::3]], axis=0)
    x2 = x2.reshape(SQ, D_MODEL)
    wq_loc = lax.dynamic_slice(Wq, (0, d * H_LOC * DH), (D_MODEL, H_LOC * DH))
    wo_loc = lax.dynamic_slice(
        Wo, (d * H_LOC * DH, 0), (H_LOC * DH, D_MODEL)
    ).reshape(H_LOC, DH, D_MODEL)

    def _perm_kv(t):
        tb = t.reshape(SKV, H_LOC, DH).transpose(1, 0, 2)
        tb = tb.reshape(H_LOC, N_MB, MB, DH)
        tb = jnp.concatenate([tb[:, 0::3], tb[:, 1::3], tb[:, 2::3]], axis=1)
        return tb.reshape(H_LOC, SKV, DH)

    k_loc = _perm_kv(K_ext)
    v_loc = _perm_kv(V_ext)

    partial = pl.pallas_call(
        _attn_body,
        in_specs=[pl.BlockSpec(memory_space=pltpu.VMEM)] * 5,
        out_specs=pl.BlockSpec(memory_space=pltpu.VMEM),
        out_shape=jax.ShapeDtypeStruct((SQ, D_MODEL), jnp.float32),
        compiler_params=pltpu.CompilerParams(
            vmem_limit_bytes=100 * 1024 * 1024
        ),
    )(x2, wq_loc, k_loc, v_loc, wo_loc)

    reduced = pl.pallas_call(
        _allreduce_body,
        in_specs=[pl.BlockSpec(memory_space=pltpu.VMEM)],
        out_specs=pl.BlockSpec(memory_space=pltpu.VMEM),
        out_shape=jax.ShapeDtypeStruct((N_DEV, CHUNK, D_MODEL), jnp.float32),
        scratch_shapes=[
            pltpu.VMEM((N_DEV - 1, CHUNK, D_MODEL), jnp.float32),
            pltpu.SemaphoreType.DMA((N_DEV - 1,)),
            pltpu.SemaphoreType.DMA((N_DEV - 1,)),
            pltpu.SemaphoreType.DMA,
        ],
        compiler_params=pltpu.CompilerParams(collective_id=0),
    )(partial.reshape(N_DEV, CHUNK, D_MODEL))

    return reduced.reshape(1, SQ, D_MODEL)


# device time: 288880 ns/iter; 1.8843x vs baseline; 1.1529x over previous
import jax
import jax.numpy as jnp
import numpy as np
from jax import lax
from jax.experimental import pallas as pl
from jax.experimental.pallas import tpu as pltpu

N_DEV = 16
SQ = 2048
SKV = 2048
D_MODEL = 1024
H_LOC = 8
DH = 128
MB = 64
N_MB = SQ // MB
CHUNK = SQ // N_DEV
SCALE = 0.08838834764831843

_QB_GROUPS = [[b for b in range(N_MB) if b % 3 == r] for r in range(3)]
_ROW_PERM = np.concatenate(
    [np.arange(b * MB, (b + 1) * MB) for g in _QB_GROUPS for b in g]
)
_KCOL_PERM = _ROW_PERM
_G_OFF = [0, 704, 1408]
_G_LEN = [704, 704, 640]
_GROUPS = [
    (0, 11, _G_OFF[0], _G_LEN[0], None),
    (704, 11, _G_OFF[2], _G_LEN[2], _G_OFF[1]),
    (1408, 10, _G_OFF[1], _G_LEN[1], _G_OFF[2]),
]


def _attn_body(x_ref, wq_ref, k_ref, v_ref, wo_ref, out_ref):
    q = jnp.dot(x_ref[...], wq_ref[...], preferred_element_type=jnp.float32)

    for r, (row_off, nb, m_off, m_len, d_off) in enumerate(_GROUPS):
        nr = nb * MB
        acc = jnp.zeros((nr, D_MODEL), jnp.float32)
        for h in range(H_LOC):
            qh = q[row_off:row_off + nr, h * DH:(h + 1) * DH]
            s_main = lax.dot_general(
                qh, k_ref[h, m_off:m_off + m_len],
                (((1,), (1,)), ((), ())),
                preferred_element_type=jnp.float32,
            ) * SCALE
            if d_off is None:
                s = s_main
            else:
                s0 = lax.dot_general(
                    qh, k_ref[h, 0:MB], (((1,), (1,)), ((), ())),
                    preferred_element_type=jnp.float32,
                ) * SCALE
                qd = qh.reshape(nb, MB, DH)
                kd = k_ref[h, d_off:d_off + nr].reshape(nb, MB, DH)
                sd = lax.dot_general(
                    qd, kd, (((2,), (2,)), ((0,), (0,))),
                    preferred_element_type=jnp.float32,
                ) * SCALE
                s = jnp.concatenate(
                    [s_main, s0, sd.reshape(nr, MB)], axis=1
                )
            m = jnp.max(s, axis=1, keepdims=True)
            w = jnp.exp(s - m)
            w = w / jnp.sum(w, axis=1, keepdims=True)
            ctx = jnp.dot(
                w[:, :m_len], v_ref[h, m_off:m_off + m_len],
                preferred_element_type=jnp.float32,
            )
            if d_off is not None:
                ctx = ctx + jnp.dot(
                    w[:, m_len:m_len + MB], v_ref[h, 0:MB],
                    preferred_element_type=jnp.float32,
                )
                wd = w[:, m_len + MB:].reshape(nb, MB, MB)
                vd = v_ref[h, d_off:d_off + nr].reshape(nb, MB, DH)
                ctx = ctx + lax.dot_general(
                    wd, vd, (((2,), (1,)), ((0,), (0,))),
                    preferred_element_type=jnp.float32,
                ).reshape(nr, DH)
            acc = acc + jnp.dot(
                ctx, wo_ref[h], preferred_element_type=jnp.float32
            )
        for j, qb in enumerate(_QB_GROUPS[r]):
            out_ref[qb * MB:(qb + 1) * MB, :] = acc[j * MB:(j + 1) * MB, :]


HALF = D_MODEL // 2


def _allreduce_body(
    p_ref, out_ref, recv_cw, recv_ccw,
    rs_cw_sems, rs_ccw_sems, ag_cw_sems, ag_ccw_sems,
    send_cw_sem, send_ccw_sem,
):
    d = lax.axis_index("i")
    left = (d - 1) % N_DEV
    right = (d + 1) % N_DEV

    barrier_sem = pltpu.get_barrier_semaphore()
    for nbr in [left, right]:
        pl.semaphore_signal(
            barrier_sem, inc=1,
            device_id=(nbr,), device_id_type=pl.DeviceIdType.MESH,
        )
    pl.semaphore_wait(barrier_sem, 2)

    out_ref[...] = p_ref[...]

    for h in range(N_DEV - 1):
        s = (d - h) % N_DEV
        cw = pltpu.make_async_remote_copy(
            src_ref=out_ref.at[s, :, 0:HALF],
            dst_ref=recv_cw.at[h],
            send_sem=send_cw_sem,
            recv_sem=rs_cw_sems.at[h],
            device_id=(right,),
            device_id_type=pl.DeviceIdType.MESH,
        )
        t = (d + h) % N_DEV
        ccw = pltpu.make_async_remote_copy(
            src_ref=out_ref.at[t, :, HALF:D_MODEL],
            dst_ref=recv_ccw.at[h],
            send_sem=send_ccw_sem,
            recv_sem=rs_ccw_sems.at[h],
            device_id=(left,),
            device_id_type=pl.DeviceIdType.MESH,
        )
        cw.start()
        ccw.start()
        cw.wait()
        ccw.wait()
        sl = (d - 1 - h) % N_DEV
        out_ref[sl, :, 0:HALF] = out_ref[sl, :, 0:HALF] + recv_cw[h]
        tl = (d + 1 + h) % N_DEV
        out_ref[tl, :, HALF:D_MODEL] = (
            out_ref[tl, :, HALF:D_MODEL] + recv_ccw[h]
        )

    for h in range(N_DEV - 1):
        s = (d + 1 - h) % N_DEV
        cw = pltpu.make_async_remote_copy(
            src_ref=out_ref.at[s, :, 0:HALF],
            dst_ref=out_ref.at[s, :, 0:HALF],
            send_sem=send_cw_sem,
            recv_sem=ag_cw_sems.at[h],
            device_id=(right,),
            device_id_type=pl.DeviceIdType.MESH,
        )
        t = (d - 1 + h) % N_DEV
        ccw = pltpu.make_async_remote_copy(
            src_ref=out_ref.at[t, :, HALF:D_MODEL],
            dst_ref=out_ref.at[t, :, HALF:D_MODEL],
            send_sem=send_ccw_sem,
            recv_sem=ag_ccw_sems.at[h],
            device_id=(left,),
            device_id_type=pl.DeviceIdType.MESH,
        )
        cw.start()
        ccw.start()
        cw.wait()
        ccw.wait()


def kernel(x, Wq, K_ext, V_ext, Wo):
    d = lax.axis_index("i")

    xb = x.reshape(N_MB, MB, D_MODEL)
    x2 = jnp.concatenate([xb[0::3], xb[1::3], xb[2::3]], axis=0)
    x2 = x2.reshape(SQ, D_MODEL)
    wq_loc = lax.dynamic_slice(Wq, (0, d * H_LOC * DH), (D_MODEL, H_LOC * DH))
    wo_loc = lax.dynamic_slice(
        Wo, (d * H_LOC * DH, 0), (H_LOC * DH, D_MODEL)
    ).reshape(H_LOC, DH, D_MODEL)

    def _perm_kv(t):
        tb = t.reshape(SKV, H_LOC, DH).transpose(1, 0, 2)
        tb = tb.reshape(H_LOC, N_MB, MB, DH)
        tb = jnp.concatenate([tb[:, 0::3], tb[:, 1::3], tb[:, 2::3]], axis=1)
        return tb.reshape(H_LOC, SKV, DH)

    k_loc = _perm_kv(K_ext)
    v_loc = _perm_kv(V_ext)

    partial = pl.pallas_call(
        _attn_body,
        in_specs=[pl.BlockSpec(memory_space=pltpu.VMEM)] * 5,
        out_specs=pl.BlockSpec(memory_space=pltpu.VMEM),
        out_shape=jax.ShapeDtypeStruct((SQ, D_MODEL), jnp.float32),
        compiler_params=pltpu.CompilerParams(
            vmem_limit_bytes=100 * 1024 * 1024
        ),
    )(x2, wq_loc, k_loc, v_loc, wo_loc)

    reduced = pl.pallas_call(
        _allreduce_body,
        in_specs=[pl.BlockSpec(memory_space=pltpu.VMEM)],
        out_specs=pl.BlockSpec(memory_space=pltpu.VMEM),
        out_shape=jax.ShapeDtypeStruct((N_DEV, CHUNK, D_MODEL), jnp.float32),
        scratch_shapes=[
            pltpu.VMEM((N_DEV - 1, CHUNK, HALF), jnp.float32),
            pltpu.VMEM((N_DEV - 1, CHUNK, HALF), jnp.float32),
            pltpu.SemaphoreType.DMA((N_DEV - 1,)),
            pltpu.SemaphoreType.DMA((N_DEV - 1,)),
            pltpu.SemaphoreType.DMA((N_DEV - 1,)),
            pltpu.SemaphoreType.DMA((N_DEV - 1,)),
            pltpu.SemaphoreType.DMA,
            pltpu.SemaphoreType.DMA,
        ],
        compiler_params=pltpu.CompilerParams(collective_id=0),
    )(partial.reshape(N_DEV, CHUNK, D_MODEL))

    return reduced.reshape(1, SQ, D_MODEL)


# device time: 247824 ns/iter; 2.1965x vs baseline; 1.1657x over previous
import jax
import jax.numpy as jnp
import numpy as np
from jax import lax
from jax.experimental import pallas as pl
from jax.experimental.pallas import tpu as pltpu

N_DEV = 16
SQ = 2048
SKV = 2048
D_MODEL = 1024
H_LOC = 8
DH = 128
MB = 64
N_MB = SQ // MB
CHUNK = SQ // N_DEV
SCALE = 0.08838834764831843

_QB_GROUPS = [[b for b in range(N_MB) if b % 3 == r] for r in range(3)]
_ROW_PERM = np.concatenate(
    [np.arange(b * MB, (b + 1) * MB) for g in _QB_GROUPS for b in g]
)
_KCOL_PERM = _ROW_PERM
_G_OFF = [0, 704, 1408]
_G_LEN = [704, 704, 640]
_GROUPS = [
    (0, 11, _G_OFF[0], _G_LEN[0], None),
    (704, 11, _G_OFF[2], _G_LEN[2], _G_OFF[1]),
    (1408, 10, _G_OFF[1], _G_LEN[1], _G_OFF[2]),
]


def _attn_body(x_ref, wq_ref, k_ref, v_ref, wo_ref, out_ref):
    q = jnp.dot(x_ref[...], wq_ref[...], preferred_element_type=jnp.float32)

    for r, (row_off, nb, m_off, m_len, d_off) in enumerate(_GROUPS):
        nr = nb * MB
        acc = jnp.zeros((nr, D_MODEL), jnp.float32)
        for h in range(H_LOC):
            qh = q[row_off:row_off + nr, h * DH:(h + 1) * DH]
            s_main = lax.dot_general(
                qh, k_ref[h, m_off:m_off + m_len],
                (((1,), (1,)), ((), ())),
                preferred_element_type=jnp.float32,
            ) * SCALE
            if d_off is None:
                s = s_main
            else:
                s0 = lax.dot_general(
                    qh, k_ref[h, 0:MB], (((1,), (1,)), ((), ())),
                    preferred_element_type=jnp.float32,
                ) * SCALE
                qd = qh.reshape(nb, MB, DH)
                kd = k_ref[h, d_off:d_off + nr].reshape(nb, MB, DH)
                sd = lax.dot_general(
                    qd, kd, (((2,), (2,)), ((0,), (0,))),
                    preferred_element_type=jnp.float32,
                ) * SCALE
                s = jnp.concatenate(
                    [s_main, s0, sd.reshape(nr, MB)], axis=1
                )
            m = jnp.max(s, axis=1, keepdims=True)
            w = jnp.exp(s - m)
            w = w / jnp.sum(w, axis=1, keepdims=True)
            ctx = jnp.dot(
                w[:, :m_len], v_ref[h, m_off:m_off + m_len],
                preferred_element_type=jnp.float32,
            )
            if d_off is not None:
                ctx = ctx + jnp.dot(
                    w[:, m_len:m_len + MB], v_ref[h, 0:MB],
                    preferred_element_type=jnp.float32,
                )
                wd = w[:, m_len + MB:].reshape(nb, MB, MB)
                vd = v_ref[h, d_off:d_off + nr].reshape(nb, MB, DH)
                ctx = ctx + lax.dot_general(
                    wd, vd, (((2,), (1,)), ((0,), (0,))),
                    preferred_element_type=jnp.float32,
                ).reshape(nr, DH)
            acc = acc + jnp.dot(
                ctx, wo_ref[h], preferred_element_type=jnp.float32
            )
        for j, qb in enumerate(_QB_GROUPS[r]):
            out_ref[qb * MB:(qb + 1) * MB, :] = acc[j * MB:(j + 1) * MB, :]


HALF = D_MODEL // 2


def _allreduce_body(
    p_ref, out_ref, recv_cw, recv_ccw, ag_recv_cw, ag_recv_ccw, sb_cw, sb_ccw,
    rs_cw_sems, rs_ccw_sems, ag_cw_sems, ag_ccw_sems,
    send_cw_sem, send_ccw_sem,
):
    d = lax.axis_index("i")
    left = (d - 1) % N_DEV
    right = (d + 1) % N_DEV

    barrier_sem = pltpu.get_barrier_semaphore()
    for nbr in [left, right]:
        pl.semaphore_signal(
            barrier_sem, inc=1,
            device_id=(nbr,), device_id_type=pl.DeviceIdType.MESH,
        )
    pl.semaphore_wait(barrier_sem, 2)

    out_ref[...] = p_ref[...]

    for h in range(N_DEV - 1):
        s = (d - h) % N_DEV
        sb_cw[...] = out_ref[s, :, 0:HALF].astype(jnp.bfloat16)
        cw = pltpu.make_async_remote_copy(
            src_ref=sb_cw,
            dst_ref=recv_cw.at[h],
            send_sem=send_cw_sem,
            recv_sem=rs_cw_sems.at[h],
            device_id=(right,),
            device_id_type=pl.DeviceIdType.MESH,
        )
        t = (d + h) % N_DEV
        sb_ccw[...] = out_ref[t, :, HALF:D_MODEL].astype(jnp.bfloat16)
        ccw = pltpu.make_async_remote_copy(
            src_ref=sb_ccw,
            dst_ref=recv_ccw.at[h],
            send_sem=send_ccw_sem,
            recv_sem=rs_ccw_sems.at[h],
            device_id=(left,),
            device_id_type=pl.DeviceIdType.MESH,
        )
        cw.start()
        ccw.start()
        cw.wait()
        ccw.wait()
        sl = (d - 1 - h) % N_DEV
        out_ref[sl, :, 0:HALF] = (
            out_ref[sl, :, 0:HALF] + recv_cw[h].astype(jnp.float32)
        )
        tl = (d + 1 + h) % N_DEV
        out_ref[tl, :, HALF:D_MODEL] = (
            out_ref[tl, :, HALF:D_MODEL] + recv_ccw[h].astype(jnp.float32)
        )

    for h in range(N_DEV - 1):
        s = (d + 1 - h) % N_DEV
        sb_cw[...] = out_ref[s, :, 0:HALF].astype(jnp.bfloat16)
        cw = pltpu.make_async_remote_copy(
            src_ref=sb_cw,
            dst_ref=ag_recv_cw.at[h],
            send_sem=send_cw_sem,
            recv_sem=ag_cw_sems.at[h],
            device_id=(right,),
            device_id_type=pl.DeviceIdType.MESH,
        )
        t = (d - 1 + h) % N_DEV
        sb_ccw[...] = out_ref[t, :, HALF:D_MODEL].astype(jnp.bfloat16)
        ccw = pltpu.make_async_remote_copy(
            src_ref=sb_ccw,
            dst_ref=ag_recv_ccw.at[h],
            send_sem=send_ccw_sem,
            recv_sem=ag_ccw_sems.at[h],
            device_id=(left,),
            device_id_type=pl.DeviceIdType.MESH,
        )
        cw.start()
        ccw.start()
        cw.wait()
        ccw.wait()
        sr = (d - h) % N_DEV
        out_ref[sr, :, 0:HALF] = ag_recv_cw[h].astype(jnp.float32)
        tr = (d + h) % N_DEV
        out_ref[tr, :, HALF:D_MODEL] = ag_recv_ccw[h].astype(jnp.float32)


def kernel(x, Wq, K_ext, V_ext, Wo):
    d = lax.axis_index("i")

    xb = x.reshape(N_MB, MB, D_MODEL)
    x2 = jnp.concatenate([xb[0::3], xb[1::3], xb[2::3]], axis=0)
    x2 = x2.reshape(SQ, D_MODEL)
    wq_loc = lax.dynamic_slice(Wq, (0, d * H_LOC * DH), (D_MODEL, H_LOC * DH))
    wo_loc = lax.dynamic_slice(
        Wo, (d * H_LOC * DH, 0), (H_LOC * DH, D_MODEL)
    ).reshape(H_LOC, DH, D_MODEL)

    def _perm_kv(t):
        tb = t.reshape(SKV, H_LOC, DH).transpose(1, 0, 2)
        tb = tb.reshape(H_LOC, N_MB, MB, DH)
        tb = jnp.concatenate([tb[:, 0::3], tb[:, 1::3], tb[:, 2::3]], axis=1)
        return tb.reshape(H_LOC, SKV, DH)

    k_loc = _perm_kv(K_ext)
    v_loc = _perm_kv(V_ext)

    partial = pl.pallas_call(
        _attn_body,
        in_specs=[pl.BlockSpec(memory_space=pltpu.VMEM)] * 5,
        out_specs=pl.BlockSpec(memory_space=pltpu.VMEM),
        out_shape=jax.ShapeDtypeStruct((SQ, D_MODEL), jnp.float32),
        compiler_params=pltpu.CompilerParams(
            vmem_limit_bytes=100 * 1024 * 1024
        ),
    )(x2, wq_loc, k_loc, v_loc, wo_loc)

    reduced = pl.pallas_call(
        _allreduce_body,
        in_specs=[pl.BlockSpec(memory_space=pltpu.VMEM)],
        out_specs=pl.BlockSpec(memory_space=pltpu.VMEM),
        out_shape=jax.ShapeDtypeStruct((N_DEV, CHUNK, D_MODEL), jnp.float32),
        scratch_shapes=[
            pltpu.VMEM((N_DEV - 1, CHUNK, HALF), jnp.bfloat16),
            pltpu.VMEM((N_DEV - 1, CHUNK, HALF), jnp.bfloat16),
            pltpu.VMEM((N_DEV - 1, CHUNK, HALF), jnp.bfloat16),
            pltpu.VMEM((N_DEV - 1, CHUNK, HALF), jnp.bfloat16),
            pltpu.VMEM((CHUNK, HALF), jnp.bfloat16),
            pltpu.VMEM((CHUNK, HALF), jnp.bfloat16),
            pltpu.SemaphoreType.DMA((N_DEV - 1,)),
            pltpu.SemaphoreType.DMA((N_DEV - 1,)),
            pltpu.SemaphoreType.DMA((N_DEV - 1,)),
            pltpu.SemaphoreType.DMA((N_DEV - 1,)),
            pltpu.SemaphoreType.DMA,
            pltpu.SemaphoreType.DMA,
        ],
        compiler_params=pltpu.CompilerParams(collective_id=0),
    )(partial.reshape(N_DEV, CHUNK, D_MODEL))

    return reduced.reshape(1, SQ, D_MODEL)


# device time: 246456 ns/iter; 2.2087x vs baseline; 1.0056x over previous
import jax
import jax.numpy as jnp
from jax import lax
from jax.experimental import pallas as pl
from jax.experimental.pallas import tpu as pltpu

N_DEV = 16
SQ = 2048
SKV = 2048
D_MODEL = 1024
H_LOC = 8
DH = 128
MB = 64
N_MB = SQ // MB
CHUNK = SQ // N_DEV
HALF = D_MODEL // 2
SCALE = 0.08838834764831843

_QB_GROUPS = [[b for b in range(N_MB) if b % 3 == r] for r in range(3)]
_G_OFF = [0, 704, 1408]
_G_LEN = [704, 704, 640]
_GROUPS = [
    (0, 11, _G_OFF[0], _G_LEN[0], None),
    (704, 11, _G_OFF[2], _G_LEN[2], _G_OFF[1]),
    (1408, 10, _G_OFF[1], _G_LEN[1], _G_OFF[2]),
]


def _body(
    x_ref, wq_ref, k_ref, v_ref, wo_ref, out_ref,
    acc_ref, recv_cw, recv_ccw,
    rs_cw_sems, rs_ccw_sems, ag_cw_sems, ag_ccw_sems,
    send_cw_sem, send_ccw_sem,
):
    q = jnp.dot(x_ref[...], wq_ref[...], preferred_element_type=jnp.float32)

    for r, (row_off, nb, m_off, m_len, d_off) in enumerate(_GROUPS):
        nr = nb * MB
        acc = jnp.zeros((nr, D_MODEL), jnp.float32)
        for h in range(H_LOC):
            qh = q[row_off:row_off + nr, h * DH:(h + 1) * DH]
            s_main = lax.dot_general(
                qh, k_ref[h, m_off:m_off + m_len],
                (((1,), (1,)), ((), ())),
                preferred_element_type=jnp.float32,
            ) * SCALE
            if d_off is None:
                s = s_main
            else:
                s0 = lax.dot_general(
                    qh, k_ref[h, 0:MB], (((1,), (1,)), ((), ())),
                    preferred_element_type=jnp.float32,
                ) * SCALE
                qd = qh.reshape(nb, MB, DH)
                kd = k_ref[h, d_off:d_off + nr].reshape(nb, MB, DH)
                sd = lax.dot_general(
                    qd, kd, (((2,), (2,)), ((0,), (0,))),
                    preferred_element_type=jnp.float32,
                ) * SCALE
                s = jnp.concatenate(
                    [s_main, s0, sd.reshape(nr, MB)], axis=1
                )
            m = jnp.max(s, axis=1, keepdims=True)
            w = jnp.exp(s - m)
            w = w / jnp.sum(w, axis=1, keepdims=True)
            ctx = jnp.dot(
                w[:, :m_len], v_ref[h, m_off:m_off + m_len],
                preferred_element_type=jnp.float32,
            )
            if d_off is not None:
                ctx = ctx + jnp.dot(
                    w[:, m_len:m_len + MB], v_ref[h, 0:MB],
                    preferred_element_type=jnp.float32,
                )
                wd = w[:, m_len + MB:].reshape(nb, MB, MB)
                vd = v_ref[h, d_off:d_off + nr].reshape(nb, MB, DH)
                ctx = ctx + lax.dot_general(
                    wd, vd, (((2,), (1,)), ((0,), (0,))),
                    preferred_element_type=jnp.float32,
                ).reshape(nr, DH)
            acc = acc + jnp.dot(
                ctx, wo_ref[h], preferred_element_type=jnp.float32
            )
        for j, qb in enumerate(_QB_GROUPS[r]):
            acc_ref[qb // 2, (qb % 2) * MB:(qb % 2 + 1) * MB, :] = (
                acc[j * MB:(j + 1) * MB, :].astype(jnp.bfloat16)
            )

    d = lax.axis_index("i")
    left = (d - 1) % N_DEV
    right = (d + 1) % N_DEV

    barrier_sem = pltpu.get_barrier_semaphore()
    for nbr in [left, right]:
        pl.semaphore_signal(
            barrier_sem, inc=1,
            device_id=(nbr,), device_id_type=pl.DeviceIdType.MESH,
        )
    pl.semaphore_wait(barrier_sem, 2)

    for h in range(N_DEV - 1):
        s = (d - h) % N_DEV
        cw = pltpu.make_async_remote_copy(
            src_ref=acc_ref.at[s, :, 0:HALF],
            dst_ref=recv_cw.at[h],
            send_sem=send_cw_sem,
            recv_sem=rs_cw_sems.at[h],
            device_id=(right,),
            device_id_type=pl.DeviceIdType.MESH,
        )
        t = (d + h) % N_DEV
        ccw = pltpu.make_async_remote_copy(
            src_ref=acc_ref.at[t, :, HALF:D_MODEL],
            dst_ref=recv_ccw.at[h],
            send_sem=send_ccw_sem,
            recv_sem=rs_ccw_sems.at[h],
            device_id=(left,),
            device_id_type=pl.DeviceIdType.MESH,
        )
        cw.start()
        ccw.start()
        cw.wait()
        ccw.wait()
        sl = (d - 1 - h) % N_DEV
        acc_ref[sl, :, 0:HALF] = acc_ref[sl, :, 0:HALF] + recv_cw[h]
        tl = (d + 1 + h) % N_DEV
        acc_ref[tl, :, HALF:D_MODEL] = (
            acc_ref[tl, :, HALF:D_MODEL] + recv_ccw[h]
        )

    for h in range(N_DEV - 1):
        s = (d + 1 - h) % N_DEV
        cw = pltpu.make_async_remote_copy(
            src_ref=acc_ref.at[s, :, 0:HALF],
            dst_ref=acc_ref.at[s, :, 0:HALF],
            send_sem=send_cw_sem,
            recv_sem=ag_cw_sems.at[h],
            device_id=(right,),
            device_id_type=pl.DeviceIdType.MESH,
        )
        t = (d - 1 + h) % N_DEV
        ccw = pltpu.make_async_remote_copy(
            src_ref=acc_ref.at[t, :, HALF:D_MODEL],
            dst_ref=acc_ref.at[t, :, HALF:D_MODEL],
            send_sem=send_ccw_sem,
            recv_sem=ag_ccw_sems.at[h],
            device_id=(left,),
            device_id_type=pl.DeviceIdType.MESH,
        )
        cw.start()
        ccw.start()
        cw.wait()
        ccw.wait()

    out_ref[...] = acc_ref[...].astype(jnp.float32)


def kernel(x, Wq, K_ext, V_ext, Wo):
    d = lax.axis_index("i")

    xb = x.reshape(N_MB, MB, D_MODEL)
    x2 = jnp.concatenate([xb[0::3], xb[1::3], xb[2::3]], axis=0)
    x2 = x2.reshape(SQ, D_MODEL)
    wq_loc = lax.dynamic_slice(Wq, (0, d * H_LOC * DH), (D_MODEL, H_LOC * DH))
    wo_loc = lax.dynamic_slice(
        Wo, (d * H_LOC * DH, 0), (H_LOC * DH, D_MODEL)
    ).reshape(H_LOC, DH, D_MODEL)

    def _perm_kv(t):
        tb = t.reshape(SKV, H_LOC, DH).transpose(1, 0, 2)
        tb = tb.reshape(H_LOC, N_MB, MB, DH)
        tb = jnp.concatenate([tb[:, 0::3], tb[:, 1::3], tb[:, 2::3]], axis=1)
        return tb.reshape(H_LOC, SKV, DH)

    k_loc = _perm_kv(K_ext)
    v_loc = _perm_kv(V_ext)

    reduced = pl.pallas_call(
        _body,
        in_specs=[pl.BlockSpec(memory_space=pltpu.VMEM)] * 5,
        out_specs=pl.BlockSpec(memory_space=pltpu.VMEM),
        out_shape=jax.ShapeDtypeStruct((N_DEV, CHUNK, D_MODEL), jnp.float32),
        scratch_shapes=[
            pltpu.VMEM((N_DEV, CHUNK, D_MODEL), jnp.bfloat16),
            pltpu.VMEM((N_DEV - 1, CHUNK, HALF), jnp.bfloat16),
            pltpu.VMEM((N_DEV - 1, CHUNK, HALF), jnp.bfloat16),
            pltpu.SemaphoreType.DMA((N_DEV - 1,)),
            pltpu.SemaphoreType.DMA((N_DEV - 1,)),
            pltpu.SemaphoreType.DMA((N_DEV - 1,)),
            pltpu.SemaphoreType.DMA((N_DEV - 1,)),
            pltpu.SemaphoreType.DMA,
            pltpu.SemaphoreType.DMA,
        ],
        compiler_params=pltpu.CompilerParams(
            collective_id=0,
            vmem_limit_bytes=100 * 1024 * 1024,
        ),
    )(x2, wq_loc, k_loc, v_loc, wo_loc)

    return reduced.reshape(1, SQ, D_MODEL)


# device time: 189432 ns/iter; 2.8735x vs baseline; 1.3010x over previous
import os

import jax
import jax.numpy as jnp
from jax import lax
from jax.experimental import pallas as pl
from jax.experimental.pallas import tpu as pltpu

_SKIP_COMM = os.environ.get("SKIP_COMM") == "1"

N_DEV = 16
SQ = 2048
SKV = 2048
D_MODEL = 1024
H_LOC = 8
DH = 128
MB = 64
N_MB = SQ // MB
CHUNK = SQ // N_DEV
HALF = D_MODEL // 2
SCALE = 0.08838834764831843

_QB_GROUPS = [[b for b in range(N_MB) if b % 3 == r] for r in range(3)]
_G_OFF = [0, 704, 1408]
_G_LEN = [704, 704, 640]
_GROUPS = [
    (0, 11, _G_OFF[0], _G_LEN[0], None),
    (704, 11, _G_OFF[2], _G_LEN[2], _G_OFF[1]),
    (1408, 10, _G_OFF[1], _G_LEN[1], _G_OFF[2]),
]


def _body(
    x_ref, wq_ref, k_ref, v_ref, wo_ref, out_ref,
    acc_ref, a_recv_cw, a_recv_ccw,
    b1_recv_lo, b1_recv_hi, b2_recv_lo, b2_recv_hi,
    a_cw_sems, a_ccw_sems,
    b1_lo_sem, b1_hi_sem, b2_lo_sem, b2_hi_sem,
    c1_lo_sem, c1_hi_sem, c2_lo_sem, c2_hi_sem,
    d_cw_sems, d_ccw_sems,
    send_cw_sem, send_ccw_sem,
):
    q = jnp.dot(x_ref[...], wq_ref[...], preferred_element_type=jnp.float32)

    for r, (row_off, nb, m_off, m_len, d_off) in enumerate(_GROUPS):
        nr = nb * MB
        acc = jnp.zeros((nr, D_MODEL), jnp.float32)
        for h in range(H_LOC):
            qh = q[row_off:row_off + nr, h * DH:(h + 1) * DH]
            s_main = lax.dot_general(
                qh, k_ref[h, m_off:m_off + m_len],
                (((1,), (1,)), ((), ())),
                preferred_element_type=jnp.float32,
            ) * SCALE
            if d_off is None:
                s = s_main
            else:
                s0 = lax.dot_general(
                    qh, k_ref[h, 0:MB], (((1,), (1,)), ((), ())),
                    preferred_element_type=jnp.float32,
                ) * SCALE
                qd = qh.reshape(nb, MB, DH)
                kd = k_ref[h, d_off:d_off + nr].reshape(nb, MB, DH)
                sd = lax.dot_general(
                    qd, kd, (((2,), (2,)), ((0,), (0,))),
                    preferred_element_type=jnp.float32,
                ) * SCALE
                s = jnp.concatenate(
                    [s_main, s0, sd.reshape(nr, MB)], axis=1
                )
            m = jnp.max(s, axis=1, keepdims=True)
            w = jnp.exp(s - m)
            w = w / jnp.sum(w, axis=1, keepdims=True)
            ctx = jnp.dot(
                w[:, :m_len], v_ref[h, m_off:m_off + m_len],
                preferred_element_type=jnp.float32,
            )
            if d_off is not None:
                ctx = ctx + jnp.dot(
                    w[:, m_len:m_len + MB], v_ref[h, 0:MB],
                    preferred_element_type=jnp.float32,
                )
                wd = w[:, m_len + MB:].reshape(nb, MB, MB)
                vd = v_ref[h, d_off:d_off + nr].reshape(nb, MB, DH)
                ctx = ctx + lax.dot_general(
                    wd, vd, (((2,), (1,)), ((0,), (0,))),
                    preferred_element_type=jnp.float32,
                ).reshape(nr, DH)
            acc = acc + jnp.dot(
                ctx, wo_ref[h], preferred_element_type=jnp.float32
            )
        for j, qb in enumerate(_QB_GROUPS[r]):
            acc_ref[qb // 2, (qb % 2) * MB:(qb % 2 + 1) * MB, :] = (
                acc[j * MB:(j + 1) * MB, :].astype(jnp.bfloat16)
            )

    if _SKIP_COMM:
        out_ref[...] = acc_ref[...].astype(jnp.float32)
        return
    d = lax.axis_index("i")
    q = d % 4
    p = d // 4
    base = d - q
    right_pl = base + (q + 1) % 4
    left_pl = base + (q - 1) % 4
    pz1 = (p + 1 - 2 * (p % 2)) * 4 + q
    pz2 = (p + 2 - 4 * ((p // 2) % 2)) * 4 + q

    barrier_sem = pltpu.get_barrier_semaphore()
    for nbr in [left_pl, right_pl, pz1, pz2]:
        pl.semaphore_signal(
            barrier_sem, inc=1,
            device_id=(nbr,), device_id_type=pl.DeviceIdType.MESH,
        )
    pl.semaphore_wait(barrier_sem, 4)

    def _copy(dst_dev, src, dst, s_sem, r_sem):
        return pltpu.make_async_remote_copy(
            src_ref=src, dst_ref=dst, send_sem=s_sem, recv_sem=r_sem,
            device_id=(dst_dev,), device_id_type=pl.DeviceIdType.MESH,
        )

    for h in range(3):
        s = ((q - h) % 4) * 4
        cw = _copy(right_pl, acc_ref.at[pl.ds(s, 4), :, 0:HALF],
                   a_recv_cw.at[h], send_cw_sem, a_cw_sems.at[h])
        t = ((q + h) % 4) * 4
        ccw = _copy(left_pl, acc_ref.at[pl.ds(t, 4), :, HALF:D_MODEL],
                    a_recv_ccw.at[h], send_ccw_sem, a_ccw_sems.at[h])
        cw.start()
        ccw.start()
        cw.wait()
        ccw.wait()
        sl = ((q - 1 - h) % 4) * 4
        acc_ref[pl.ds(sl, 4), :, 0:HALF] = (
            acc_ref[pl.ds(sl, 4), :, 0:HALF] + a_recv_cw[h]
        )
        tl = ((q + 1 + h) % 4) * 4
        acc_ref[pl.ds(tl, 4), :, HALF:D_MODEL] = (
            acc_ref[pl.ds(tl, 4), :, HALF:D_MODEL] + a_recv_ccw[h]
        )

    qlo = ((q + 1) % 4) * 4
    qhi = ((q - 1) % 4) * 4
    keep2 = 2 * (p // 2)
    sent2 = 2 - keep2
    keep1 = p % 2
    sent1 = 1 - keep1

    b_lo = _copy(pz2, acc_ref.at[pl.ds(qlo + sent2, 2), :, 0:HALF],
                 b1_recv_lo, send_cw_sem, b1_lo_sem)
    b_hi = _copy(pz2, acc_ref.at[pl.ds(qhi + sent2, 2), :, HALF:D_MODEL],
                 b1_recv_hi, send_ccw_sem, b1_hi_sem)
    b_lo.start()
    b_hi.start()
    b_lo.wait()
    b_hi.wait()
    acc_ref[pl.ds(qlo + keep2, 2), :, 0:HALF] = (
        acc_ref[pl.ds(qlo + keep2, 2), :, 0:HALF] + b1_recv_lo[...]
    )
    acc_ref[pl.ds(qhi + keep2, 2), :, HALF:D_MODEL] = (
        acc_ref[pl.ds(qhi + keep2, 2), :, HALF:D_MODEL] + b1_recv_hi[...]
    )

    b_lo = _copy(pz1, acc_ref.at[qlo + keep2 + sent1, :, 0:HALF],
                 b2_recv_lo, send_cw_sem, b2_lo_sem)
    b_hi = _copy(pz1, acc_ref.at[qhi + keep2 + sent1, :, HALF:D_MODEL],
                 b2_recv_hi, send_ccw_sem, b2_hi_sem)
    b_lo.start()
    b_hi.start()
    b_lo.wait()
    b_hi.wait()
    clo = qlo + keep2 + keep1
    chi = qhi + keep2 + keep1
    acc_ref[clo, :, 0:HALF] = acc_ref[clo, :, 0:HALF] + b2_recv_lo[...]
    acc_ref[chi, :, HALF:D_MODEL] = (
        acc_ref[chi, :, HALF:D_MODEL] + b2_recv_hi[...]
    )

    c_lo = _copy(pz1, acc_ref.at[clo, :, 0:HALF],
                 acc_ref.at[clo, :, 0:HALF], send_cw_sem, c1_lo_sem)
    c_hi = _copy(pz1, acc_ref.at[chi, :, HALF:D_MODEL],
                 acc_ref.at[chi, :, HALF:D_MODEL], send_ccw_sem, c1_hi_sem)
    c_lo.start()
    c_hi.start()
    c_lo.wait()
    c_hi.wait()

    c_lo = _copy(pz2, acc_ref.at[pl.ds(qlo + keep2, 2), :, 0:HALF],
                 acc_ref.at[pl.ds(qlo + keep2, 2), :, 0:HALF],
                 send_cw_sem, c2_lo_sem)
    c_hi = _copy(pz2, acc_ref.at[pl.ds(qhi + keep2, 2), :, HALF:D_MODEL],
                 acc_ref.at[pl.ds(qhi + keep2, 2), :, HALF:D_MODEL],
                 send_ccw_sem, c2_hi_sem)
    c_lo.start()
    c_hi.start()
    c_lo.wait()
    c_hi.wait()

    for h in range(3):
        s = ((q + 1 - h) % 4) * 4
        cw = _copy(right_pl, acc_ref.at[pl.ds(s, 4), :, 0:HALF],
                   acc_ref.at[pl.ds(s, 4), :, 0:HALF],
                   send_cw_sem, d_cw_sems.at[h])
        t = ((q - 1 + h) % 4) * 4
        ccw = _copy(left_pl, acc_ref.at[pl.ds(t, 4), :, HALF:D_MODEL],
                    acc_ref.at[pl.ds(t, 4), :, HALF:D_MODEL],
                    send_ccw_sem, d_ccw_sems.at[h])
        cw.start()
        ccw.start()
        cw.wait()
        ccw.wait()

    out_ref[...] = acc_ref[...].astype(jnp.float32)


def kernel(x, Wq, K_ext, V_ext, Wo):
    d = lax.axis_index("i")

    xb = x.reshape(N_MB, MB, D_MODEL)
    x2 = jnp.concatenate([xb[0::3], xb[1::3], xb[2::3]], axis=0)
    x2 = x2.reshape(SQ, D_MODEL)
    wq_loc = lax.dynamic_slice(Wq, (0, d * H_LOC * DH), (D_MODEL, H_LOC * DH))
    wo_loc = lax.dynamic_slice(
        Wo, (d * H_LOC * DH, 0), (H_LOC * DH, D_MODEL)
    ).reshape(H_LOC, DH, D_MODEL)

    def _perm_kv(t):
        tb = t.reshape(SKV, H_LOC, DH).transpose(1, 0, 2)
        tb = tb.reshape(H_LOC, N_MB, MB, DH)
        tb = jnp.concatenate([tb[:, 0::3], tb[:, 1::3], tb[:, 2::3]], axis=1)
        return tb.reshape(H_LOC, SKV, DH)

    k_loc = _perm_kv(K_ext)
    v_loc = _perm_kv(V_ext)

    reduced = pl.pallas_call(
        _body,
        in_specs=[pl.BlockSpec(memory_space=pltpu.VMEM)] * 5,
        out_specs=pl.BlockSpec(memory_space=pltpu.VMEM),
        out_shape=jax.ShapeDtypeStruct((N_DEV, CHUNK, D_MODEL), jnp.float32),
        scratch_shapes=[
            pltpu.VMEM((N_DEV, CHUNK, D_MODEL), jnp.bfloat16),
            pltpu.VMEM((3, 4, CHUNK, HALF), jnp.bfloat16),
            pltpu.VMEM((3, 4, CHUNK, HALF), jnp.bfloat16),
            pltpu.VMEM((2, CHUNK, HALF), jnp.bfloat16),
            pltpu.VMEM((2, CHUNK, HALF), jnp.bfloat16),
            pltpu.VMEM((CHUNK, HALF), jnp.bfloat16),
            pltpu.VMEM((CHUNK, HALF), jnp.bfloat16),
            pltpu.SemaphoreType.DMA((3,)),
            pltpu.SemaphoreType.DMA((3,)),
            pltpu.SemaphoreType.DMA,
            pltpu.SemaphoreType.DMA,
            pltpu.SemaphoreType.DMA,
            pltpu.SemaphoreType.DMA,
            pltpu.SemaphoreType.DMA,
            pltpu.SemaphoreType.DMA,
            pltpu.SemaphoreType.DMA,
            pltpu.SemaphoreType.DMA,
            pltpu.SemaphoreType.DMA((3,)),
            pltpu.SemaphoreType.DMA((3,)),
            pltpu.SemaphoreType.DMA,
            pltpu.SemaphoreType.DMA,
        ],
        compiler_params=pltpu.CompilerParams(
            **({} if _SKIP_COMM else {"collective_id": 0}),
            vmem_limit_bytes=100 * 1024 * 1024,
        ),
    )(x2, wq_loc, k_loc, v_loc, wo_loc)

    return reduced.reshape(1, SQ, D_MODEL)


# device time: 179183 ns/iter; 3.0379x vs baseline; 1.0572x over previous
import os

import jax
import jax.numpy as jnp
from jax import lax
from jax.experimental import pallas as pl
from jax.experimental.pallas import tpu as pltpu

_SKIP_COMM = os.environ.get("SKIP_COMM") == "1"

N_DEV = 16
SQ = 2048
SKV = 2048
D_MODEL = 1024
H_LOC = 8
DH = 128
MB = 64
N_MB = SQ // MB
CHUNK = SQ // N_DEV
HALF = D_MODEL // 2
SCALE = 0.08838834764831843

_QB_GROUPS = [[b for b in range(N_MB) if b % 3 == r] for r in range(3)]
_G_OFF = [0, 704, 1408]
_G_LEN = [704, 704, 640]
_GROUPS = [
    (0, 11, _G_OFF[0], _G_LEN[0], None),
    (704, 11, _G_OFF[2], _G_LEN[2], _G_OFF[1]),
    (1408, 10, _G_OFF[1], _G_LEN[1], _G_OFF[2]),
]


def _body(
    x_ref, wq_hbm, k_ref, v_ref, wo_hbm, out_ref,
    wq_loc, wo_loc, wq_sem, wo_sem,
    acc_ref, a_recv_cw, a_recv_ccw,
    b1_recv_lo, b1_recv_hi, b2_recv_lo, b2_recv_hi,
    a_cw_sems, a_ccw_sems,
    b1_lo_sem, b1_hi_sem, b2_lo_sem, b2_hi_sem,
    c1_lo_sem, c1_hi_sem, c2_lo_sem, c2_hi_sem,
    d_cw_sems, d_ccw_sems,
    send_cw_sem, send_ccw_sem,
):
    dd = lax.axis_index("i")
    wq_dma = pltpu.make_async_copy(
        wq_hbm.at[:, pl.ds(dd * H_LOC * DH, H_LOC * DH)], wq_loc, wq_sem
    )
    wq_dma.start()
    wo_dma = pltpu.make_async_copy(
        wo_hbm.at[pl.ds(dd * H_LOC * DH, H_LOC * DH), :], wo_loc, wo_sem
    )
    wo_dma.start()
    wq_dma.wait()

    q = jnp.dot(x_ref[...], wq_loc[...], preferred_element_type=jnp.float32)
    wo_dma.wait()

    for r, (row_off, nb, m_off, m_len, d_off) in enumerate(_GROUPS):
        nr = nb * MB
        acc = jnp.zeros((nr, D_MODEL), jnp.float32)
        for h in range(H_LOC):
            qh = q[row_off:row_off + nr, h * DH:(h + 1) * DH]
            s_main = lax.dot_general(
                qh, k_ref[h, m_off:m_off + m_len],
                (((1,), (1,)), ((), ())),
                preferred_element_type=jnp.float32,
            ) * SCALE
            if d_off is None:
                s = s_main
            else:
                s0 = lax.dot_general(
                    qh, k_ref[h, 0:MB], (((1,), (1,)), ((), ())),
                    preferred_element_type=jnp.float32,
                ) * SCALE
                qd = qh.reshape(nb, MB, DH)
                kd = k_ref[h, d_off:d_off + nr].reshape(nb, MB, DH)
                sd = lax.dot_general(
                    qd, kd, (((2,), (2,)), ((0,), (0,))),
                    preferred_element_type=jnp.float32,
                ) * SCALE
                s = jnp.concatenate(
                    [s_main, s0, sd.reshape(nr, MB)], axis=1
                )
            m = jnp.max(s, axis=1, keepdims=True)
            w = jnp.exp(s - m)
            w = w / jnp.sum(w, axis=1, keepdims=True)
            ctx = jnp.dot(
                w[:, :m_len], v_ref[h, m_off:m_off + m_len],
                preferred_element_type=jnp.float32,
            )
            if d_off is not None:
                ctx = ctx + jnp.dot(
                    w[:, m_len:m_len + MB], v_ref[h, 0:MB],
                    preferred_element_type=jnp.float32,
                )
                wd = w[:, m_len + MB:].reshape(nb, MB, MB)
                vd = v_ref[h, d_off:d_off + nr].reshape(nb, MB, DH)
                ctx = ctx + lax.dot_general(
                    wd, vd, (((2,), (1,)), ((0,), (0,))),
                    preferred_element_type=jnp.float32,
                ).reshape(nr, DH)
            acc = acc + jnp.dot(
                ctx, wo_loc[h * DH:(h + 1) * DH, :],
                preferred_element_type=jnp.float32,
            )
        for j, qb in enumerate(_QB_GROUPS[r]):
            acc_ref[qb // 2, (qb % 2) * MB:(qb % 2 + 1) * MB, :] = (
                acc[j * MB:(j + 1) * MB, :].astype(jnp.bfloat16)
            )

    if _SKIP_COMM:
        out_ref[...] = acc_ref[...].astype(jnp.float32)
        return
    d = lax.axis_index("i")
    q = d % 4
    p = d // 4
    base = d - q
    right_pl = base + (q + 1) % 4
    left_pl = base + (q - 1) % 4
    pz1 = (p + 1 - 2 * (p % 2)) * 4 + q
    pz2 = (p + 2 - 4 * ((p // 2) % 2)) * 4 + q

    barrier_sem = pltpu.get_barrier_semaphore()
    for nbr in [left_pl, right_pl, pz1, pz2]:
        pl.semaphore_signal(
            barrier_sem, inc=1,
            device_id=(nbr,), device_id_type=pl.DeviceIdType.MESH,
        )
    pl.semaphore_wait(barrier_sem, 4)

    def _copy(dst_dev, src, dst, s_sem, r_sem):
        return pltpu.make_async_remote_copy(
            src_ref=src, dst_ref=dst, send_sem=s_sem, recv_sem=r_sem,
            device_id=(dst_dev,), device_id_type=pl.DeviceIdType.MESH,
        )

    for h in range(3):
        s = ((q - h) % 4) * 4
        cw = _copy(right_pl, acc_ref.at[pl.ds(s, 4), :, 0:HALF],
                   a_recv_cw.at[h], send_cw_sem, a_cw_sems.at[h])
        t = ((q + h) % 4) * 4
        ccw = _copy(left_pl, acc_ref.at[pl.ds(t, 4), :, HALF:D_MODEL],
                    a_recv_ccw.at[h], send_ccw_sem, a_ccw_sems.at[h])
        cw.start()
        ccw.start()
        cw.wait()
        ccw.wait()
        sl = ((q - 1 - h) % 4) * 4
        acc_ref[pl.ds(sl, 4), :, 0:HALF] = (
            acc_ref[pl.ds(sl, 4), :, 0:HALF] + a_recv_cw[h]
        )
        tl = ((q + 1 + h) % 4) * 4
        acc_ref[pl.ds(tl, 4), :, HALF:D_MODEL] = (
            acc_ref[pl.ds(tl, 4), :, HALF:D_MODEL] + a_recv_ccw[h]
        )

    qlo = ((q + 1) % 4) * 4
    qhi = ((q - 1) % 4) * 4
    keep2 = 2 * (p // 2)
    sent2 = 2 - keep2
    keep1 = p % 2
    sent1 = 1 - keep1

    b_lo = _copy(pz2, acc_ref.at[pl.ds(qlo + sent2, 2), :, 0:HALF],
                 b1_recv_lo, send_cw_sem, b1_lo_sem)
    b_hi = _copy(pz2, acc_ref.at[pl.ds(qhi + sent2, 2), :, HALF:D_MODEL],
                 b1_recv_hi, send_ccw_sem, b1_hi_sem)
    b_lo.start()
    b_hi.start()
    b_lo.wait()
    b_hi.wait()
    acc_ref[pl.ds(qlo + keep2, 2), :, 0:HALF] = (
        acc_ref[pl.ds(qlo + keep2, 2), :, 0:HALF] + b1_recv_lo[...]
    )
    acc_ref[pl.ds(qhi + keep2, 2), :, HALF:D_MODEL] = (
        acc_ref[pl.ds(qhi + keep2, 2), :, HALF:D_MODEL] + b1_recv_hi[...]
    )

    b_lo = _copy(pz1, acc_ref.at[qlo + keep2 + sent1, :, 0:HALF],
                 b2_recv_lo, send_cw_sem, b2_lo_sem)
    b_hi = _copy(pz1, acc_ref.at[qhi + keep2 + sent1, :, HALF:D_MODEL],
                 b2_recv_hi, send_ccw_sem, b2_hi_sem)
    b_lo.start()
    b_hi.start()
    b_lo.wait()
    b_hi.wait()
    clo = qlo + keep2 + keep1
    chi = qhi + keep2 + keep1
    acc_ref[clo, :, 0:HALF] = acc_ref[clo, :, 0:HALF] + b2_recv_lo[...]
    acc_ref[chi, :, HALF:D_MODEL] = (
        acc_ref[chi, :, HALF:D_MODEL] + b2_recv_hi[...]
    )

    c_lo = _copy(pz1, acc_ref.at[clo, :, 0:HALF],
                 acc_ref.at[clo, :, 0:HALF], send_cw_sem, c1_lo_sem)
    c_hi = _copy(pz1, acc_ref.at[chi, :, HALF:D_MODEL],
                 acc_ref.at[chi, :, HALF:D_MODEL], send_ccw_sem, c1_hi_sem)
    c_lo.start()
    c_hi.start()
    c_lo.wait()
    c_hi.wait()

    c_lo = _copy(pz2, acc_ref.at[pl.ds(qlo + keep2, 2), :, 0:HALF],
                 acc_ref.at[pl.ds(qlo + keep2, 2), :, 0:HALF],
                 send_cw_sem, c2_lo_sem)
    c_hi = _copy(pz2, acc_ref.at[pl.ds(qhi + keep2, 2), :, HALF:D_MODEL],
                 acc_ref.at[pl.ds(qhi + keep2, 2), :, HALF:D_MODEL],
                 send_ccw_sem, c2_hi_sem)
    c_lo.start()
    c_hi.start()
    c_lo.wait()
    c_hi.wait()

    for h in range(3):
        s = ((q + 1 - h) % 4) * 4
        cw = _copy(right_pl, acc_ref.at[pl.ds(s, 4), :, 0:HALF],
                   acc_ref.at[pl.ds(s, 4), :, 0:HALF],
                   send_cw_sem, d_cw_sems.at[h])
        t = ((q - 1 + h) % 4) * 4
        ccw = _copy(left_pl, acc_ref.at[pl.ds(t, 4), :, HALF:D_MODEL],
                    acc_ref.at[pl.ds(t, 4), :, HALF:D_MODEL],
                    send_ccw_sem, d_ccw_sems.at[h])
        cw.start()
        ccw.start()
        cw.wait()
        ccw.wait()

    out_ref[...] = acc_ref[...].astype(jnp.float32)


def kernel(x, Wq, K_ext, V_ext, Wo):
    d = lax.axis_index("i")

    xb = x.reshape(N_MB, MB, D_MODEL)
    x2 = jnp.concatenate([xb[0::3], xb[1::3], xb[2::3]], axis=0)
    x2 = x2.reshape(SQ, D_MODEL)

    def _perm_kv(t):
        tb = t.reshape(SKV, H_LOC, DH).transpose(1, 0, 2)
        tb = tb.reshape(H_LOC, N_MB, MB, DH)
        tb = jnp.concatenate([tb[:, 0::3], tb[:, 1::3], tb[:, 2::3]], axis=1)
        return tb.reshape(H_LOC, SKV, DH)

    k_loc = _perm_kv(K_ext)
    v_loc = _perm_kv(V_ext)

    reduced = pl.pallas_call(
        _body,
        in_specs=[
            pl.BlockSpec(memory_space=pltpu.VMEM),
            pl.BlockSpec(memory_space=pl.ANY),
            pl.BlockSpec(memory_space=pltpu.VMEM),
            pl.BlockSpec(memory_space=pltpu.VMEM),
            pl.BlockSpec(memory_space=pl.ANY),
        ],
        out_specs=pl.BlockSpec(memory_space=pltpu.VMEM),
        out_shape=jax.ShapeDtypeStruct((N_DEV, CHUNK, D_MODEL), jnp.float32),
        scratch_shapes=[
            pltpu.VMEM((D_MODEL, H_LOC * DH), jnp.float32),
            pltpu.VMEM((H_LOC * DH, D_MODEL), jnp.float32),
            pltpu.SemaphoreType.DMA,
            pltpu.SemaphoreType.DMA,
            pltpu.VMEM((N_DEV, CHUNK, D_MODEL), jnp.bfloat16),
            pltpu.VMEM((3, 4, CHUNK, HALF), jnp.bfloat16),
            pltpu.VMEM((3, 4, CHUNK, HALF), jnp.bfloat16),
            pltpu.VMEM((2, CHUNK, HALF), jnp.bfloat16),
            pltpu.VMEM((2, CHUNK, HALF), jnp.bfloat16),
            pltpu.VMEM((CHUNK, HALF), jnp.bfloat16),
            pltpu.VMEM((CHUNK, HALF), jnp.bfloat16),
            pltpu.SemaphoreType.DMA((3,)),
            pltpu.SemaphoreType.DMA((3,)),
            pltpu.SemaphoreType.DMA,
            pltpu.SemaphoreType.DMA,
            pltpu.SemaphoreType.DMA,
            pltpu.SemaphoreType.DMA,
            pltpu.SemaphoreType.DMA,
            pltpu.SemaphoreType.DMA,
            pltpu.SemaphoreType.DMA,
            pltpu.SemaphoreType.DMA,
            pltpu.SemaphoreType.DMA((3,)),
            pltpu.SemaphoreType.DMA((3,)),
            pltpu.SemaphoreType.DMA,
            pltpu.SemaphoreType.DMA,
        ],
        compiler_params=pltpu.CompilerParams(
            **({} if _SKIP_COMM else {"collective_id": 0}),
            vmem_limit_bytes=100 * 1024 * 1024,
        ),
    )(x2, Wq, k_loc, v_loc, Wo)

    return reduced.reshape(1, SQ, D_MODEL)
